# baseline (speedup 1.0000x reference)
import jax
import jax.numpy as jnp
from jax.experimental import pallas as pl
from jax.experimental.pallas import tpu as pltpu

N = 10000
F_IN = 128
F_OUT = 128
BM = 256


def _gcn_kernel(x_ref, adj_ref, deg_ref, w_ref, b_ref, out_ref, support_ref):
    i = pl.program_id(0)

    @pl.when(i == 0)
    def _():
        support_ref[...] = (
            jnp.dot(x_ref[...], w_ref[...], preferred_element_type=jnp.float32)
            + b_ref[...]
        )

    agg = jnp.dot(adj_ref[...], support_ref[...], preferred_element_type=jnp.float32)
    out_ref[...] = jnp.maximum(agg * deg_ref[...], 0.0)


@jax.jit
def kernel(x, adj_matrix, degree_norm, W, b):
    b2 = b.reshape(1, F_OUT)
    grid = (pl.cdiv(N, BM),)
    return pl.pallas_call(
        _gcn_kernel,
        grid=grid,
        in_specs=[
            pl.BlockSpec((N, F_IN), lambda i: (0, 0)),
            pl.BlockSpec((BM, N), lambda i: (i, 0)),
            pl.BlockSpec((BM, 1), lambda i: (i, 0)),
            pl.BlockSpec((F_IN, F_OUT), lambda i: (0, 0)),
            pl.BlockSpec((1, F_OUT), lambda i: (0, 0)),
        ],
        out_specs=pl.BlockSpec((BM, F_OUT), lambda i: (i, 0)),
        out_shape=jax.ShapeDtypeStruct((N, F_OUT), jnp.float32),
        scratch_shapes=[pltpu.VMEM((N, F_OUT), jnp.float32)],
        compiler_params=pltpu.CompilerParams(
            dimension_semantics=("arbitrary",),
        ),
    )(x, adj_matrix, degree_norm, W, b2)


# speedup vs baseline: 1.0135x; 1.0135x over previous
import jax
import jax.numpy as jnp
from jax.experimental import pallas as pl
from jax.experimental.pallas import tpu as pltpu

N = 10000
F_IN = 128
F_OUT = 128
BM = 224


def _gcn_kernel(x_ref, adj_ref, deg_ref, w_ref, b_ref, out_ref, support_ref):
    i = pl.program_id(0)

    @pl.when(i == 0)
    def _():
        support_ref[...] = (
            jnp.dot(x_ref[...], w_ref[...], preferred_element_type=jnp.float32)
            + b_ref[...]
        )

    agg = jnp.dot(adj_ref[...], support_ref[...], preferred_element_type=jnp.float32)
    out_ref[...] = jnp.maximum(agg * deg_ref[...], 0.0)


@jax.jit
def kernel(x, adj_matrix, degree_norm, W, b):
    b2 = b.reshape(1, F_OUT)
    grid = (pl.cdiv(N, BM),)
    return pl.pallas_call(
        _gcn_kernel,
        grid=grid,
        in_specs=[
            pl.BlockSpec((N, F_IN), lambda i: (0, 0)),
            pl.BlockSpec((BM, N), lambda i: (i, 0)),
            pl.BlockSpec((BM, 1), lambda i: (i, 0)),
            pl.BlockSpec((F_IN, F_OUT), lambda i: (0, 0)),
            pl.BlockSpec((1, F_OUT), lambda i: (0, 0)),
        ],
        out_specs=pl.BlockSpec((BM, F_OUT), lambda i: (i, 0)),
        out_shape=jax.ShapeDtypeStruct((N, F_OUT), jnp.float32),
        scratch_shapes=[pltpu.VMEM((N, F_OUT), jnp.float32)],
        compiler_params=pltpu.CompilerParams(
            dimension_semantics=("arbitrary",),
        ),
    )(x, adj_matrix, degree_norm, W, b2)
